# baseline (speedup 1.0000x reference)
import functools

import jax
import jax.numpy as jnp
from jax import lax
from jax.experimental import pallas as pl
from jax.experimental.pallas import tpu as pltpu
from jax.experimental.pallas import tpu_sc as plsc

N_NODES = 100000
N_RELS = 1000
EMB = 200
B = 16384

ROW = EMB * 2
PROW = 512
NVEC = ROW // 16
NC = 2
NS = 16
NW = NC * NS
PER_W = B // NW
CHUNK = 32
NCHUNK = PER_W // CHUNK

_GDN = lax.GatherDimensionNumbers(
    offset_dims=(), collapsed_slice_dims=(0,), start_index_map=(0,))


def _perm(x, idx):
    return lax.gather(x, idx[:, None], dimension_numbers=_GDN,
                      slice_sizes=(1,),
                      mode=lax.GatherScatterMode.PROMISE_IN_BOUNDS)


def _rsqrt(x):
    xi = lax.bitcast_convert_type(x, jnp.int32)
    yi = jnp.int32(0x5F3759DF) - (xi >> 1)
    y = lax.bitcast_convert_type(yi, jnp.float32)
    hx = x * jnp.float32(0.5)
    for _ in range(3):
        y = y * (jnp.float32(1.5) - hx * y * y)
    return y


def _merge(a, b, s, lane):
    pa = _perm(a, lane ^ s)
    pb = _perm(b, lane ^ s)
    return jnp.where((lane & s) == 0, a + pa, b + pb)


def _mesh():
    return plsc.VectorSubcoreMesh(core_axis_name="c", subcore_axis_name="s")


def _worker_id():
    return lax.axis_index("s") * NC + lax.axis_index("c")


@functools.partial(
    pl.kernel,
    mesh=_mesh(),
    out_type=jax.ShapeDtypeStruct((N_RELS * PROW,), jnp.float32),
    compiler_params=pltpu.CompilerParams(use_tc_tiling_on_sc=False),
    scratch_types=[pltpu.VMEM((ROW,), jnp.float32)],
)
def _normalize_rel(rel_hbm, out_hbm, buf):
    lane = lax.iota(jnp.int32, 16)
    swap_idx = lane ^ 1
    w = _worker_id()
    rows_per_w = 32
    start = w * rows_per_w
    nrows = jnp.minimum(jnp.int32(rows_per_w), jnp.int32(N_RELS) - start)

    def body(rr, carry):
        row = start + rr
        pltpu.sync_copy(rel_hbm.at[row], buf)
        for j in range(NVEC):
            rv = buf[pl.ds(j * 16, 16)]
            sw = _perm(rv, swap_idx)
            abs2 = rv * rv + sw * sw
            inv = jnp.minimum(_rsqrt(abs2), jnp.float32(1e9))
            buf[pl.ds(j * 16, 16)] = rv * inv
        pltpu.sync_copy(buf, out_hbm.at[pl.ds(row * PROW, ROW)])
        return carry
    lax.fori_loop(0, nrows, body, jnp.int32(0))


@functools.partial(
    pl.kernel,
    mesh=_mesh(),
    out_type=jax.ShapeDtypeStruct((B,), jnp.float32),
    compiler_params=pltpu.CompilerParams(needs_layout_passes=False),
    scratch_types=[
        pltpu.VMEM((CHUNK,), jnp.int32),
        pltpu.VMEM((CHUNK,), jnp.int32),
        pltpu.VMEM((CHUNK,), jnp.int32),
        pltpu.VMEM((4 * CHUNK,), jnp.int32),
        pltpu.VMEM((4 * CHUNK,), jnp.int32),
        pltpu.VMEM((4 * CHUNK,), jnp.int32),
        pltpu.VMEM((4 * CHUNK, 128), jnp.float32),
        pltpu.VMEM((4 * CHUNK, 128), jnp.float32),
        pltpu.VMEM((4 * CHUNK, 128), jnp.float32),
        pltpu.VMEM((CHUNK * 16,), jnp.float32),
        pltpu.VMEM((PER_W,), jnp.float32),
        pltpu.SemaphoreType.DMA,
    ],
)
def _rotate_score(s_idx_hbm, r_idx_hbm, o_idx_hbm, ent_hbm, reln_hbm,
                  out_hbm, s_iv, r_iv, o_iv, s_i4, r_i4, o_i4,
                  s_4, r_4, o_4, accbuf, scores, sem):
    lane = lax.iota(jnp.int32, 16)
    swap_idx = lane ^ 1
    even_idx = lane & jnp.int32(-2)
    odd_idx = lane | jnp.int32(1)
    altsign = jnp.where((lane & 1) == 0, jnp.float32(-1.0), jnp.float32(1.0))

    w = _worker_id()
    base = w * PER_W

    def chunk_body(c, carry):
        cb = base + c * CHUNK
        pltpu.sync_copy(s_idx_hbm.at[pl.ds(cb, CHUNK)], s_iv)
        pltpu.sync_copy(r_idx_hbm.at[pl.ds(cb, CHUNK)], r_iv)
        pltpu.sync_copy(o_idx_hbm.at[pl.ds(cb, CHUNK)], o_iv)

        for iv, i4 in ((s_iv, s_i4), (r_iv, r_i4), (o_iv, o_i4)):
            for v in range(CHUNK // 16):
                r0 = iv[pl.ds(v * 16, 16)] << 2
                pos0 = (lane << 2) + jnp.int32(64 * v)
                for t in range(4):
                    plsc.store_scatter(i4, [pos0 + jnp.int32(t)],
                                       r0 + jnp.int32(t))

        cs = pltpu.async_copy(ent_hbm.at[s_i4], s_4, sem)
        cr = pltpu.async_copy(reln_hbm.at[r_i4], r_4, sem)
        co = pltpu.async_copy(ent_hbm.at[o_i4], o_4, sem)
        cs.wait()
        cr.wait()
        co.wait()

        def body(e, carry2):
            e4 = e * 4
            acc = jnp.zeros((16,), jnp.float32)
            for j in range(NVEC):
                rr = j // 8
                cc = (j % 8) * 16
                sv = s_4[e4 + rr, pl.ds(cc, 16)]
                rv = r_4[e4 + rr, pl.ds(cc, 16)]
                ov = o_4[e4 + rr, pl.ds(cc, 16)]
                ssw = _perm(sv, swap_idx)
                ra = _perm(rv, even_idx)
                rb = _perm(rv, odd_idx) * altsign
                rot = sv * ra + ssw * rb
                d = rot - ov
                acc = acc + d * d
            accbuf[pl.ds(e * 16, 16)] = acc
            return carry2
        lax.fori_loop(0, CHUNK, body, jnp.int32(0))

        def reduce_body(g, carry2):
            gb = g * 256
            vs = [accbuf[pl.ds(gb + i * 16, 16)] for i in range(16)]
            for s in (1, 2, 4, 8):
                vs = [_merge(vs[i], vs[i + 1], s, lane)
                      for i in range(0, len(vs), 2)]
            tot = vs[0]
            y = _rsqrt(jnp.maximum(tot, jnp.float32(1e-38)))
            scores[pl.ds(c * CHUNK + g * 16, 16)] = tot * y
            return carry2
        lax.fori_loop(0, CHUNK // 16, reduce_body, jnp.int32(0))
        return carry

    lax.fori_loop(0, NCHUNK, chunk_body, jnp.int32(0))
    pltpu.sync_copy(scores, out_hbm.at[pl.ds(base, PER_W)])


def kernel(s_idx, r_idx, o_idx, ent, rel):
    s_idx = s_idx.astype(jnp.int32)
    r_idx = r_idx.astype(jnp.int32)
    o_idx = o_idx.astype(jnp.int32)
    rel_n = _normalize_rel(rel).reshape(N_RELS * 4, 128)
    ent4 = jnp.pad(ent, ((0, 0), (0, PROW - ROW))).reshape(N_NODES * 4, 128)
    return _rotate_score(s_idx, r_idx, o_idx, ent4, rel_n)


# speedup vs baseline: 4.5472x; 4.5472x over previous
import functools

import jax
import jax.numpy as jnp
from jax import lax
from jax.experimental import pallas as pl
from jax.experimental.pallas import tpu as pltpu
from jax.experimental.pallas import tpu_sc as plsc

N_NODES = 100000
N_RELS = 1000
EMB = 200
B = 16384

ROW = EMB * 2
NVEC = ROW // 16
NC = 2
NS = 16
NW = NC * NS
PER_W = B // NW
CHUNK = 64
NCHUNK = PER_W // CHUNK

_GDN = lax.GatherDimensionNumbers(
    offset_dims=(), collapsed_slice_dims=(0,), start_index_map=(0,))


def _perm(x, idx):
    return lax.gather(x, idx[:, None], dimension_numbers=_GDN,
                      slice_sizes=(1,),
                      mode=lax.GatherScatterMode.PROMISE_IN_BOUNDS)


def _rsqrt(x):
    xi = lax.bitcast_convert_type(x, jnp.int32)
    yi = jnp.int32(0x5F3759DF) - (xi >> 1)
    y = lax.bitcast_convert_type(yi, jnp.float32)
    hx = x * jnp.float32(0.5)
    for _ in range(3):
        y = y * (jnp.float32(1.5) - hx * y * y)
    return y


def _merge(a, b, s, lane):
    pa = _perm(a, lane ^ s)
    pb = _perm(b, lane ^ s)
    return jnp.where((lane & s) == 0, a + pa, b + pb)


def _mesh():
    return plsc.VectorSubcoreMesh(core_axis_name="c", subcore_axis_name="s")


def _worker_id():
    return lax.axis_index("s") * NC + lax.axis_index("c")


@functools.partial(
    pl.kernel,
    mesh=_mesh(),
    out_type=jax.ShapeDtypeStruct((N_RELS, ROW), jnp.float32),
    compiler_params=pltpu.CompilerParams(use_tc_tiling_on_sc=False),
    scratch_types=[pltpu.VMEM((32, ROW), jnp.float32)],
)
def _normalize_rel(rel_hbm, out_hbm, buf):
    lane = lax.iota(jnp.int32, 16)
    swap_idx = lane ^ 1
    w = _worker_id()
    tail = N_RELS - 31 * 32

    def process(nrows):
        def body(r, carry):
            for j in range(NVEC):
                rv = buf[r, pl.ds(j * 16, 16)]
                sw = _perm(rv, swap_idx)
                abs2 = rv * rv + sw * sw
                inv = jnp.minimum(_rsqrt(abs2), jnp.float32(1e9))
                buf[r, pl.ds(j * 16, 16)] = rv * inv
            return carry
        lax.fori_loop(0, nrows, body, jnp.int32(0))

    @pl.when(w < 31)
    def _():
        pltpu.sync_copy(rel_hbm.at[pl.ds(w * 32, 32)], buf)
        process(32)
        pltpu.sync_copy(buf, out_hbm.at[pl.ds(w * 32, 32)])

    @pl.when(w == 31)
    def _():
        pltpu.sync_copy(rel_hbm.at[pl.ds(31 * 32, tail)], buf.at[pl.ds(0, tail)])
        process(tail)
        pltpu.sync_copy(buf.at[pl.ds(0, tail)], out_hbm.at[pl.ds(31 * 32, tail)])


@functools.partial(
    pl.kernel,
    mesh=_mesh(),
    out_type=jax.ShapeDtypeStruct((B,), jnp.float32),
    compiler_params=pltpu.CompilerParams(needs_layout_passes=False),
    scratch_types=[
        pltpu.VMEM((CHUNK,), jnp.int32),
        pltpu.VMEM((CHUNK,), jnp.int32),
        pltpu.VMEM((CHUNK,), jnp.int32),
        pltpu.VMEM((CHUNK, ROW), jnp.float32),
        pltpu.VMEM((CHUNK, ROW), jnp.float32),
        pltpu.VMEM((CHUNK, ROW), jnp.float32),
        pltpu.VMEM((CHUNK * 16,), jnp.float32),
        pltpu.VMEM((PER_W,), jnp.float32),
        pltpu.SemaphoreType.DMA,
    ],
)
def _rotate_score(s_idx_hbm, r_idx_hbm, o_idx_hbm, ent_hbm, reln_hbm,
                  out_hbm, s_iv, r_iv, o_iv,
                  s_rows, r_rows, o_rows, accbuf, scores, sem):
    lane = lax.iota(jnp.int32, 16)
    swap_idx = lane ^ 1
    even_idx = lane & jnp.int32(-2)
    odd_idx = lane | jnp.int32(1)
    altsign = jnp.where((lane & 1) == 0, jnp.float32(-1.0), jnp.float32(1.0))

    w = _worker_id()
    base = w * PER_W

    def chunk_body(c, carry):
        cb = base + c * CHUNK
        pltpu.sync_copy(s_idx_hbm.at[pl.ds(cb, CHUNK)], s_iv)
        pltpu.sync_copy(r_idx_hbm.at[pl.ds(cb, CHUNK)], r_iv)
        pltpu.sync_copy(o_idx_hbm.at[pl.ds(cb, CHUNK)], o_iv)
        def fire(g, carry2):
            sv16 = s_iv[pl.ds(g * 16, 16)]
            rv16 = r_iv[pl.ds(g * 16, 16)]
            ov16 = o_iv[pl.ds(g * 16, 16)]
            e0 = g * 16
            for l in range(16):
                pltpu.async_copy(ent_hbm.at[sv16[l]], s_rows.at[e0 + l], sem)
                pltpu.async_copy(reln_hbm.at[rv16[l]], r_rows.at[e0 + l], sem)
                pltpu.async_copy(ent_hbm.at[ov16[l]], o_rows.at[e0 + l], sem)
            return carry2
        lax.fori_loop(0, CHUNK // 16, fire, jnp.int32(0))

        def drain(e, carry2):
            pltpu.make_async_copy(ent_hbm.at[0], s_rows.at[e], sem).wait()
            pltpu.make_async_copy(ent_hbm.at[0], r_rows.at[e], sem).wait()
            pltpu.make_async_copy(ent_hbm.at[0], o_rows.at[e], sem).wait()
            return carry2
        lax.fori_loop(0, CHUNK, drain, jnp.int32(0))

        def body(e, carry2):
            acc = jnp.zeros((16,), jnp.float32)
            for j in range(NVEC):
                sv = s_rows[e, pl.ds(j * 16, 16)]
                rv = r_rows[e, pl.ds(j * 16, 16)]
                ov = o_rows[e, pl.ds(j * 16, 16)]
                ssw = _perm(sv, swap_idx)
                ra = _perm(rv, even_idx)
                rb = _perm(rv, odd_idx) * altsign
                rot = sv * ra + ssw * rb
                d = rot - ov
                acc = acc + d * d
            accbuf[pl.ds(e * 16, 16)] = acc
            return carry2
        lax.fori_loop(0, CHUNK, body, jnp.int32(0))

        def reduce_body(g, carry2):
            gb = g * 256
            vs = [accbuf[pl.ds(gb + i * 16, 16)] for i in range(16)]
            for s in (1, 2, 4, 8):
                vs = [_merge(vs[i], vs[i + 1], s, lane)
                      for i in range(0, len(vs), 2)]
            tot = vs[0]
            y = _rsqrt(jnp.maximum(tot, jnp.float32(1e-38)))
            scores[pl.ds(c * CHUNK + g * 16, 16)] = tot * y
            return carry2
        lax.fori_loop(0, CHUNK // 16, reduce_body, jnp.int32(0))
        return carry

    lax.fori_loop(0, NCHUNK, chunk_body, jnp.int32(0))
    pltpu.sync_copy(scores, out_hbm.at[pl.ds(base, PER_W)])


def kernel(s_idx, r_idx, o_idx, ent, rel):
    s_idx = s_idx.astype(jnp.int32)
    r_idx = r_idx.astype(jnp.int32)
    o_idx = o_idx.astype(jnp.int32)
    rel_n = _normalize_rel(rel)
    return _rotate_score(s_idx, r_idx, o_idx, ent, rel_n)
